# Initial kernel scaffold; baseline (speedup 1.0000x reference)
import jax
import jax.numpy as jnp
from jax.experimental import pallas as pl


def kernel(positions, types, object_positions, object_colors, object_shapes, object_materials, object_sizes, scene_state, questions, Q_table, P_table, T_table, C_table, SH_table, M_table, SZ_table, Wp, bp, Ws, bs, Wr, br, gamma, beta):
    raise NotImplementedError("write your pallas kernel here")


# speedup vs baseline: 5.8335x; 5.8335x over previous
import functools

import jax
import jax.numpy as jnp
from jax import lax
from jax.experimental import pallas as pl
from jax.experimental.pallas import tpu as pltpu
from jax.experimental.pallas import tpu_sc as plsc

B = 1024
O = 50
QL = 200
SL = O + 1 + QL
H = 128
E = 64
QV = 100000
NPOS = 251
NT = 4
NC = 8
NS = 3
NM = 2
NZ = 2
NP = 3
NSC = 128

_NW = 32
_PW = (B * QL) // _NW
_CH = 128
_NCH = _PW // _CH


@functools.lru_cache(maxsize=1)
def _make_qgather():
    mesh = plsc.VectorSubcoreMesh(core_axis_name="c", subcore_axis_name="s")

    @functools.partial(
        pl.kernel,
        mesh=mesh,
        out_type=jax.ShapeDtypeStruct((B * QL, H), jnp.float32),
        scratch_types=[
            pltpu.VMEM((_PW,), jnp.int32),
            pltpu.VMEM((_CH, H), jnp.float32),
            pltpu.VMEM((_CH, H), jnp.float32),
            pltpu.SemaphoreType.DMA,
            pltpu.SemaphoreType.DMA,
        ],
    )
    def qgather(table_hbm, idx_hbm, out_hbm, idx_v, buf0, buf1, sem0, sem1):
        wid = lax.axis_index("s") * 2 + lax.axis_index("c")
        base = wid * _PW
        pltpu.sync_copy(idx_hbm.at[pl.ds(base, _PW)], idx_v)

        def body(j, carry):
            c0 = 2 * j
            c1 = c0 + 1
            cp0 = pltpu.async_copy(
                table_hbm.at[idx_v.at[pl.ds(c0 * _CH, _CH)]], buf0, sem0)
            cp1 = pltpu.async_copy(
                table_hbm.at[idx_v.at[pl.ds(c1 * _CH, _CH)]], buf1, sem1)
            cp0.wait()
            pltpu.sync_copy(buf0, out_hbm.at[pl.ds(base + c0 * _CH, _CH)])
            cp1.wait()
            pltpu.sync_copy(buf1, out_hbm.at[pl.ds(base + c1 * _CH, _CH)])
            return carry

        lax.fori_loop(0, _NCH // 2, body, 0)

    return qgather


_BB = 16
_NB = B // _BB


def _tc_body(pos_ref, typ_ref, opos_ref, ocol_ref, osha_ref, omat_ref,
             osiz_ref, scene_ref, qg_ref, PT_ref, C_ref, SH_ref, M_ref,
             SZ_ref, Wp_ref, bp_ref, Ws_ref, bs_ref, Wr_ref, br_ref,
             gamma_ref, beta_ref, emb_ref, mask_ref, objm_ref):
    f32 = jnp.float32
    typ = typ_ref[...]
    pos = pos_ref[...]

    mask_ref[...] = jnp.where(typ >= 1, 0.0, -10000.0).astype(f32)
    objm_ref[...] = (typ == 1).astype(f32)

    r = _BB * SL
    cols3 = lax.broadcasted_iota(jnp.int32, (_BB, SL, NPOS + NT), 2)
    oh3 = ((pos[:, :, None] == cols3)
           | ((typ[:, :, None] + NPOS) == cols3)).astype(f32)
    aug = jnp.dot(oh3.reshape(r, NPOS + NT), PT_ref[...],
                  preferred_element_type=f32)
    aug = aug.reshape(_BB, SL, H)

    wr = Wr_ref[...]
    wp_w = jnp.dot(Wp_ref[...], wr[0:E], preferred_element_type=f32)
    c_w = jnp.dot(C_ref[...], wr[E:2 * E], preferred_element_type=f32)
    sh_w = jnp.dot(SH_ref[...], wr[2 * E:3 * E], preferred_element_type=f32)
    m_w = jnp.dot(M_ref[...], wr[3 * E:4 * E], preferred_element_type=f32)
    sz_w = jnp.dot(SZ_ref[...], wr[4 * E:5 * E], preferred_element_type=f32)
    const_row = (jnp.dot(bp_ref[...], wr[0:E], preferred_element_type=f32)
                 + br_ref[...])

    opos = opos_ref[...]
    ore = jnp.zeros((_BB, O, H), f32) + const_row[0][None, None, :]
    for k in range(NP):
        ore = ore + opos[:, :, k:k + 1] * wp_w[k][None, None, :]
    ocol = ocol_ref[...][:, :, None]
    for c in range(NC):
        ore = ore + (ocol == c).astype(f32) * c_w[c][None, None, :]
    osha = osha_ref[...][:, :, None]
    for s in range(NS):
        ore = ore + (osha == s).astype(f32) * sh_w[s][None, None, :]
    omat = omat_ref[...][:, :, None]
    for m in range(NM):
        ore = ore + (omat == m).astype(f32) * m_w[m][None, None, :]
    osiz = osiz_ref[...][:, :, None]
    for z in range(NZ):
        ore = ore + (osiz == z).astype(f32) * sz_w[z][None, None, :]

    scene = scene_ref[...].reshape(_BB, NSC)
    ss = (jnp.dot(scene, Ws_ref[...], preferred_element_type=f32)
          + bs_ref[...])

    pre = jnp.concatenate([ore, ss[:, None, :], qg_ref[...]], axis=1)
    x = pre + aug

    mu = jnp.mean(x, axis=-1, keepdims=True)
    d = x - mu
    var = jnp.mean(d * d, axis=-1, keepdims=True)
    g = gamma_ref[...][0][None, None, :]
    bb = beta_ref[...][0][None, None, :]
    emb_ref[...] = d * lax.rsqrt(var + 1e-12) * g + bb


@functools.lru_cache(maxsize=1)
def _make_tc():
    bspec = pl.BlockSpec
    in_specs = [
        bspec((_BB, SL), lambda i: (i, 0)),
        bspec((_BB, SL), lambda i: (i, 0)),
        bspec((_BB, O, NP), lambda i: (i, 0, 0)),
        bspec((_BB, O), lambda i: (i, 0)),
        bspec((_BB, O), lambda i: (i, 0)),
        bspec((_BB, O), lambda i: (i, 0)),
        bspec((_BB, O), lambda i: (i, 0)),
        bspec((_BB, 1, NSC), lambda i: (i, 0, 0)),
        bspec((_BB, QL, H), lambda i: (i, 0, 0)),
        bspec((NPOS + NT, H), lambda i: (0, 0)),
        bspec((NC, E), lambda i: (0, 0)),
        bspec((NS, E), lambda i: (0, 0)),
        bspec((NM, E), lambda i: (0, 0)),
        bspec((NZ, E), lambda i: (0, 0)),
        bspec((NP, E), lambda i: (0, 0)),
        bspec((1, E), lambda i: (0, 0)),
        bspec((NSC, H), lambda i: (0, 0)),
        bspec((1, H), lambda i: (0, 0)),
        bspec((5 * E, H), lambda i: (0, 0)),
        bspec((1, H), lambda i: (0, 0)),
        bspec((1, H), lambda i: (0, 0)),
        bspec((1, H), lambda i: (0, 0)),
    ]
    out_specs = [
        bspec((_BB, SL, H), lambda i: (i, 0, 0)),
        bspec((_BB, SL), lambda i: (i, 0)),
        bspec((_BB, SL), lambda i: (i, 0)),
    ]
    out_shape = [
        jax.ShapeDtypeStruct((B, SL, H), jnp.float32),
        jax.ShapeDtypeStruct((B, SL), jnp.float32),
        jax.ShapeDtypeStruct((B, SL), jnp.float32),
    ]
    return pl.pallas_call(
        _tc_body,
        grid=(_NB,),
        in_specs=in_specs,
        out_specs=out_specs,
        out_shape=out_shape,
    )


def kernel(positions, types, object_positions, object_colors, object_shapes,
           object_materials, object_sizes, scene_state, questions, Q_table,
           P_table, T_table, C_table, SH_table, M_table, SZ_table, Wp, bp,
           Ws, bs, Wr, br, gamma, beta):
    q1d = questions.reshape(B * QL)
    qg = _make_qgather()(Q_table, q1d)
    qg3 = qg.reshape(B, QL, H)
    pt = jnp.concatenate([P_table, T_table], axis=0)
    emb, mask2d, objm = _make_tc()(
        positions, types, object_positions, object_colors, object_shapes,
        object_materials, object_sizes, scene_state, qg3, pt, C_table,
        SH_table, M_table, SZ_table, Wp, bp.reshape(1, E), Ws,
        bs.reshape(1, H), Wr, br.reshape(1, H), gamma.reshape(1, H),
        beta.reshape(1, H))
    return emb, mask2d.reshape(B, 1, 1, SL), objm
